# baseline (speedup 1.0000x reference)
import jax
import jax.numpy as jnp
from jax import lax
from jax.experimental import pallas as pl
from jax.experimental.pallas import tpu as pltpu
from jax.experimental.pallas import tpu_sc as plsc

_SPLIT = 256
_TAIL = 128
_TBLK = 800
_G1 = 48
_G2 = 8


def _tail_table(table, dim):
    vocab = table.shape[0]

    def body(t_ref, o_ref):
        tail = t_ref[:, _SPLIT:dim]
        o_ref[...] = jnp.concatenate(
            [tail, jnp.zeros((_TBLK, _TAIL - (dim - _SPLIT)), jnp.float32)], axis=1
        )

    return pl.pallas_call(
        body,
        grid=(vocab // _TBLK,),
        in_specs=[pl.BlockSpec((_TBLK, table.shape[1]), lambda i: (i, 0))],
        out_specs=pl.BlockSpec((_TBLK, _TAIL), lambda i: (i, 0)),
        out_shape=jax.ShapeDtypeStruct((vocab, _TAIL), jnp.float32),
    )(table)


def kernel(sentence, table):
    batch, seq = sentence.shape
    vocab, dim = table.shape
    seq_p = _G1 + 2 * _G2
    sent = jnp.pad(sentence.astype(jnp.int32), ((0, 0), (0, seq_p - seq))).reshape(-1)
    tail_tab = _tail_table(table, dim)

    info = plsc.get_sparse_core_info()
    nw = info.num_cores * info.num_subcores
    per_w = batch // nw
    assert per_w % 2 == 0
    slab = per_w * seq_p

    mesh = plsc.VectorSubcoreMesh(core_axis_name="core", subcore_axis_name="subcore")

    @pl.kernel(
        out_type=jax.ShapeDtypeStruct((batch, seq, dim), table.dtype),
        mesh=mesh,
        scratch_types=[
            pltpu.VMEM((slab,), jnp.int32),
            pltpu.VMEM((seq, dim), jnp.float32),
            pltpu.VMEM((seq, dim), jnp.float32),
            pltpu.VMEM((_G2, dim), jnp.float32),
            pltpu.VMEM((_G2, dim), jnp.float32),
            pltpu.VMEM((_G1, _TAIL), jnp.float32),
            pltpu.VMEM((_G1, _TAIL), jnp.float32),
            pltpu.VMEM((_G2, _TAIL), jnp.float32),
            pltpu.VMEM((_G2, _TAIL), jnp.float32),
            pltpu.SemaphoreType.DMA,
            pltpu.SemaphoreType.DMA,
            pltpu.SemaphoreType.DMA,
            pltpu.SemaphoreType.DMA,
        ],
    )
    def gather_kernel(
        tab_hbm, tail_hbm, idx_hbm, out_hbm,
        idx_v, rows_a, rows_b, r2_a, r2_b, tv_a, tv_b, t2_a, t2_b,
        sem_a, sem_b, wsem_a, wsem_b,
    ):
        wid = lax.axis_index("subcore") * info.num_cores + lax.axis_index("core")
        base = wid * per_w
        tab_main = tab_hbm.at[:, pl.ds(0, _SPLIT)]
        pltpu.sync_copy(idx_hbm.at[pl.ds(base * seq_p, slab)], idx_v)

        def issue(c, rows, r2, tv, t2, sem):
            i1 = idx_v.at[pl.ds(c * seq_p, _G1)]
            i2 = idx_v.at[pl.ds(c * seq_p + _G1, _G2)]
            return (
                pltpu.async_copy(tab_main.at[i1], rows.at[pl.ds(0, _G1), pl.ds(0, _SPLIT)], sem),
                pltpu.async_copy(tab_main.at[i2], r2.at[:, pl.ds(0, _SPLIT)], sem),
                pltpu.async_copy(tail_hbm.at[i1], tv, sem),
                pltpu.async_copy(tail_hbm.at[i2], t2, sem),
            )

        def finish(c, rows, r2, tv, t2, handles, wsem):
            for h in handles:
                h.wait()

            @pl.loop(0, _G1, step=8)
            def _(j0):
                for dj in range(8):
                    j = j0 + dj
                    rows[j, pl.ds(_SPLIT, 16)] = tv[j, pl.ds(0, 16)]
                    rows[j, pl.ds(_SPLIT + 16, 16)] = tv[j, pl.ds(16, 16)]
                    rows[j, pl.ds(_SPLIT + 32, 12)] = tv[j, pl.ds(32, 12)]

            for k in range(seq - _G1):
                for c16 in range(_SPLIT // 16):
                    rows[_G1 + k, pl.ds(16 * c16, 16)] = r2[k, pl.ds(16 * c16, 16)]
                rows[_G1 + k, pl.ds(_SPLIT, 16)] = t2[k, pl.ds(0, 16)]
                rows[_G1 + k, pl.ds(_SPLIT + 16, 16)] = t2[k, pl.ds(16, 16)]
                rows[_G1 + k, pl.ds(_SPLIT + 32, 12)] = t2[k, pl.ds(32, 12)]

            pltpu.async_copy(rows, out_hbm.at[base + c], wsem)

        def wb_wait(rows, c, wsem):
            pltpu.make_async_copy(rows, out_hbm.at[base + c], wsem).wait()

        @pl.loop(0, per_w, step=2)
        def _(c):
            @pl.when(c > 0)
            def _():
                wb_wait(rows_a, c - 2, wsem_a)
                wb_wait(rows_b, c - 1, wsem_b)

            ha = issue(c, rows_a, r2_a, tv_a, t2_a, sem_a)
            hb = issue(c + 1, rows_b, r2_b, tv_b, t2_b, sem_b)
            finish(c, rows_a, r2_a, tv_a, t2_a, ha, wsem_a)
            finish(c + 1, rows_b, r2_b, tv_b, t2_b, hb, wsem_b)

        wb_wait(rows_a, per_w - 2, wsem_a)
        wb_wait(rows_b, per_w - 1, wsem_b)

    return gather_kernel(table, tail_tab, sent)


# speedup vs baseline: 2.4870x; 2.4870x over previous
import jax
import jax.numpy as jnp
from jax import lax
from jax.experimental import pallas as pl
from jax.experimental.pallas import tpu as pltpu
from jax.experimental.pallas import tpu_sc as plsc

_WINDOW = 128
_UNROLL = 16


def kernel(sentence, table):
    batch, seq = sentence.shape
    vocab, dim = table.shape
    n = batch * seq
    idx = sentence.reshape(n).astype(jnp.int32)

    info = plsc.get_sparse_core_info()
    nw = info.num_cores * info.num_subcores
    per_w = n // nw
    steps = per_w // _WINDOW
    assert steps % 2 == 0

    mesh = plsc.VectorSubcoreMesh(core_axis_name="core", subcore_axis_name="subcore")

    @pl.kernel(
        out_type=jax.ShapeDtypeStruct((n, dim), table.dtype),
        mesh=mesh,
        scratch_types=[
            pltpu.VMEM((per_w,), jnp.int32),
            pltpu.VMEM((_WINDOW, dim), jnp.float32),
            pltpu.VMEM((_WINDOW, dim), jnp.float32),
            pltpu.SemaphoreType.DMA,
            pltpu.SemaphoreType.DMA,
        ],
    )
    def gather_kernel(tab_hbm, idx_hbm, out_hbm, idx_v, rows_a, rows_b, sem_a, sem_b):
        wid = lax.axis_index("subcore") * info.num_cores + lax.axis_index("core")
        base = wid * per_w
        pltpu.sync_copy(idx_hbm.at[pl.ds(base, per_w)], idx_v)

        def issue(w, rows, sem):
            @pl.loop(0, _WINDOW, step=_UNROLL)
            def _(j):
                v = idx_v[pl.ds(w * _WINDOW + j, _UNROLL)]
                for k in range(_UNROLL):
                    pltpu.async_copy(tab_hbm.at[v[k]], rows.at[j + k], sem)

        def drain_writeback(w, rows, sem):
            pltpu.make_async_copy(tab_hbm.at[pl.ds(0, _WINDOW)], rows, sem).wait()
            pltpu.sync_copy(rows, out_hbm.at[pl.ds(base + w * _WINDOW, _WINDOW)])

        @pl.loop(0, steps, step=2)
        def _(w):
            issue(w, rows_a, sem_a)
            issue(w + 1, rows_b, sem_b)
            drain_writeback(w, rows_a, sem_a)
            drain_writeback(w + 1, rows_b, sem_b)

    out = gather_kernel(table, idx)
    return out.reshape(batch, seq, dim)
